# Initial kernel scaffold; baseline (speedup 1.0000x reference)
import jax
import jax.numpy as jnp
from jax.experimental import pallas as pl


def kernel(x, params, edge_index, batch):
    raise NotImplementedError("write your pallas kernel here")


# speedup vs baseline: 1.5997x; 1.5997x over previous
import functools

import jax
import jax.numpy as jnp
from jax import lax
from jax.experimental import pallas as pl
from jax.experimental.pallas import tpu as pltpu
from jax.experimental.pallas import tpu_sc as plsc

N = 10000
E = 320000
D = 128
H = 128
G = 8
NUM_ACTIONS = 64
NUM_LAYERS = 4
RESIDUAL_START = 2
NEG_SLOPE = 0.01
EPS = 1e-5

NS = 16
CHUNK = 128
CH_PER_TILE = 160
EPT = CHUNK * CH_PER_TILE
EPAD = EPT * NS
EROWS = EPAD // CHUNK
NPAD = 10240
NHALF = NPAD // 2
RPT = NHALF // NS
CW = 16
CH_BUF = 80
IGN = -1

_f32 = jnp.float32
_HI = lax.Precision.HIGHEST


def _leaky(v):
    return jnp.where(v >= 0, v, NEG_SLOPE * v)


def _make_sc_dir():
    mesh = plsc.VectorSubcoreMesh(core_axis_name="c", subcore_axis_name="s",
                                  num_cores=2, num_subcores=NS)

    out_type = [jax.ShapeDtypeStruct((NPAD, D), _f32)]

    scratch = [
        pltpu.VMEM((CH_BUF, CHUNK), jnp.int32),
        pltpu.VMEM((CH_BUF, CHUNK), jnp.int32),
        pltpu.VMEM((CHUNK,), jnp.int32),
        pltpu.VMEM((CHUNK, D), _f32),
        pltpu.VMEM((CHUNK, D), _f32),
        pltpu.SemaphoreType.DMA,
        pltpu.SemaphoreType.DMA,
        pltpu.VMEM_SHARED((NHALF, D), _f32),
    ]

    def body(h_hbm, gidx2, sidx2, z_d,
             sum_o, gidx, sidx, sloc, rows0, rows1, sem0, sem1, acc):
        c = lax.axis_index("c")
        s = lax.axis_index("s")
        nbase = c * NHALF

        pltpu.sync_copy(z_d, acc.at[pl.ds(s * RPT, RPT)])

        plsc.subcore_barrier()

        def fire(j, buf, sem):
            pltpu.async_copy(h_hbm.at[gidx.at[j]], buf, sem)

        def wait(buf, sem):
            pltpu.make_async_copy(h_hbm.at[gidx.at[0]], buf, sem).wait()

        def scat(j, buf):
            for k in range(CHUNK // 16):
                t = sidx[j, pl.ds(k * 16, 16)]
                ok = jnp.logical_and(t >= nbase, t < nbase + NHALF)
                sloc[pl.ds(k * 16, 16)] = jnp.where(ok, t - nbase, IGN)
            idx = plsc.Indices(sloc, ignored_value=IGN)
            pltpu.sync_copy(buf, acc.at[idx], add=True)

        base = s * CH_PER_TILE
        for half in range(CH_PER_TILE // CH_BUF):
            hb = base + half * CH_BUF
            pltpu.sync_copy(gidx2.at[pl.ds(hb, CH_BUF)], gidx)
            pltpu.sync_copy(sidx2.at[pl.ds(hb, CH_BUF)], sidx)

            fire(0, rows0, sem0)
            fire(1, rows1, sem1)

            def step(i, carry):
                wait(rows0, sem0)
                scat(2 * i, rows0)
                fire(2 * i + 2, rows0, sem0)
                wait(rows1, sem1)
                scat(2 * i + 1, rows1)
                fire(2 * i + 3, rows1, sem1)
                return carry

            lax.fori_loop(0, CH_BUF // 2 - 1, step, 0)
            last = CH_BUF - 2
            wait(rows0, sem0)
            scat(last, rows0)
            wait(rows1, sem1)
            scat(last + 1, rows1)

        plsc.subcore_barrier()

        loc = pl.ds(s * RPT, RPT)
        glob = pl.ds(nbase + s * RPT, RPT)
        pltpu.sync_copy(acc.at[loc], sum_o.at[glob])

    return pl.kernel(body, out_type=out_type, mesh=mesh,
                     scratch_types=scratch)


def _make_sc_cnt():
    mesh = plsc.VectorSubcoreMesh(core_axis_name="c", subcore_axis_name="s",
                                  num_cores=2, num_subcores=NS)

    out_type = [jax.ShapeDtypeStruct((NPAD, D), _f32)]

    scratch = [
        pltpu.VMEM((CH_PER_TILE, CHUNK), jnp.int32),
        pltpu.VMEM((CHUNK,), jnp.int32),
        pltpu.VMEM((CHUNK, D), _f32),
        pltpu.VMEM_SHARED((NHALF, D), _f32),
    ]

    def body(sidx2, z_d, o_d, cnt_o, sidx, sloc, ones_v, cnt):
        c = lax.axis_index("c")
        s = lax.axis_index("s")
        nbase = c * NHALF

        pltpu.sync_copy(z_d, cnt.at[pl.ds(s * RPT, RPT)])
        pltpu.sync_copy(o_d, ones_v)
        base = s * CH_PER_TILE
        pltpu.sync_copy(sidx2.at[pl.ds(base, CH_PER_TILE)], sidx)

        plsc.subcore_barrier()

        def step(j, carry):
            for k in range(CHUNK // 16):
                t = sidx[j, pl.ds(k * 16, 16)]
                ok = jnp.logical_and(t >= nbase, t < nbase + NHALF)
                sloc[pl.ds(k * 16, 16)] = jnp.where(ok, t - nbase, IGN)
            idx = plsc.Indices(sloc, ignored_value=IGN)
            pltpu.sync_copy(ones_v, cnt.at[idx], add=True)
            return carry

        lax.fori_loop(0, CH_PER_TILE, step, 0)

        plsc.subcore_barrier()

        loc = pl.ds(s * RPT, RPT)
        glob = pl.ds(nbase + s * RPT, RPT)
        pltpu.sync_copy(cnt.at[loc], cnt_o.at[glob])

    return pl.kernel(body, out_type=out_type, mesh=mesh,
                     scratch_types=scratch)


_sc_cache = {}


def _sc_aggregate(h, src2, dst2, z_d, o_d, with_counts):
    if "dir" not in _sc_cache:
        _sc_cache["dir"] = _make_sc_dir()
        _sc_cache["cnt"] = _make_sc_cnt()
    sf, = _sc_cache["dir"](h, src2, dst2, z_d)
    sb, = _sc_cache["dir"](h, dst2, src2, z_d)
    if with_counts:
        cf, = _sc_cache["cnt"](dst2, z_d, o_d)
        cb, = _sc_cache["cnt"](src2, z_d, o_d)
        return sf, sb, cf, cb
    return sf, sb


BR = 512
NB = NPAD // BR


def _layer_a_body(h, sf, sb, cf, cb, wlt, wrt, bl, p,
                  hnew, s1, s2, cg):
    icd = 1.0 / jnp.maximum(cf[...], 1.0)
    ics = 1.0 / jnp.maximum(cb[...], 1.0)
    aggc = 0.5 * (sf[...] * icd + sb[...] * ics)
    hn = (jnp.dot(aggc, wlt[...], preferred_element_type=_f32)
          + jnp.dot(h[...], wrt[...], preferred_element_type=_f32)
          + bl[0:1, :])
    hnew[...] = hn
    pt = p[...]
    dn = (((0,), (0,)), ((), ()))
    s1c = lax.dot_general(pt, hn, dn, preferred_element_type=_f32,
                          precision=_HI)
    s2c = lax.dot_general(pt, hn * hn, dn, preferred_element_type=_f32,
                          precision=_HI)
    cgc = lax.dot_general(pt, jnp.ones_like(hn), dn,
                          preferred_element_type=_f32, precision=_HI)

    @pl.when(pl.program_id(0) == 0)
    def _():
        s1[...] = s1c
        s2[...] = s2c
        cg[...] = cgc

    @pl.when(pl.program_id(0) > 0)
    def _():
        s1[...] += s1c
        s2[...] += s2c
        cg[...] += cgc


def _layer_a(h, sf, sb, cf, cb, wlt, wrt, bl8, p):
    grid = (NB,)
    bs_rows = pl.BlockSpec((BR, D), lambda b: (b, 0))
    bs_w = pl.BlockSpec((D, D), lambda b: (0, 0))
    bs_g = pl.BlockSpec((G, D), lambda b: (0, 0))
    bs_p = pl.BlockSpec((BR, G), lambda b: (b, 0))
    return pl.pallas_call(
        _layer_a_body,
        grid=grid,
        in_specs=[bs_rows, bs_rows, bs_rows, bs_rows, bs_rows,
                  bs_w, bs_w, bs_g, bs_p],
        out_specs=[bs_rows, bs_g, bs_g, bs_g],
        out_shape=[
            jax.ShapeDtypeStruct((NPAD, D), _f32),
            jax.ShapeDtypeStruct((G, D), _f32),
            jax.ShapeDtypeStruct((G, D), _f32),
            jax.ShapeDtypeStruct((G, D), _f32),
        ],
    )(h, sf, sb, cf, cb, wlt, wrt, bl8, p)


def _make_layer_b_body(residual: bool):
    def body(*args):
        if residual:
            (hnew, hprev, s1, s2, cg, p, w8, b8, ms8, hout) = args
        else:
            (hnew, s1, s2, cg, p, w8, b8, ms8, hout) = args
        cgv = jnp.maximum(cg[...], 1.0)
        mean = s1[...] / cgv
        ms = ms8[...]
        var = s2[...] / cgv - mean * mean * ms * (2.0 - ms)
        rstd = lax.rsqrt(var + EPS)
        scale = w8[...] * rstd
        shift = b8[...] - scale * ms * mean
        rowscale = jnp.dot(p[...], scale, preferred_element_type=_f32,
                           precision=_HI)
        rowshift = jnp.dot(p[...], shift, preferred_element_type=_f32,
                           precision=_HI)
        v = _leaky(hnew[...] * rowscale + rowshift)
        if residual:
            v = v + hprev[...]
        hout[...] = v
    return body


def _layer_b(hnew, hprev, s1, s2, cg, p, w8, b8, ms8, residual):
    grid = (NB,)
    bs_rows = pl.BlockSpec((BR, D), lambda b: (b, 0))
    bs_g = pl.BlockSpec((G, D), lambda b: (0, 0))
    bs_p = pl.BlockSpec((BR, G), lambda b: (b, 0))
    in_specs = [bs_rows] + ([bs_rows] if residual else []) + \
               [bs_g, bs_g, bs_g, bs_p, bs_g, bs_g, bs_g]
    args = (hnew,) + ((hprev,) if residual else ()) + \
           (s1, s2, cg, p, w8, b8, ms8)
    return pl.pallas_call(
        _make_layer_b_body(residual),
        grid=grid,
        in_specs=in_specs,
        out_specs=bs_rows,
        out_shape=jax.ShapeDtypeStruct((NPAD, D), _f32),
    )(*args)


def _pool_heads_body(h, p, twt, tb, aw1, ab1, aw2, ab2, cw1, cb1, cw2, cb2,
                     logits, value8, acc_e, acc_c):
    b = pl.program_id(0)
    pt = p[...]
    dn = (((0,), (0,)), ((), ()))
    e = lax.dot_general(pt, h[...], dn, preferred_element_type=_f32,
                        precision=_HI)
    c = lax.dot_general(pt, jnp.ones_like(h[...]), dn,
                        preferred_element_type=_f32, precision=_HI)

    @pl.when(b == 0)
    def _():
        acc_e[...] = e
        acc_c[...] = c

    @pl.when(b > 0)
    def _():
        acc_e[...] += e
        acc_c[...] += c

    @pl.when(b == NB - 1)
    def _():
        emb = acc_e[...] / jnp.maximum(acc_c[...], 1.0)
        t = _leaky(jnp.dot(emb, twt[...], preferred_element_type=_f32)
                   + tb[...])
        a = _leaky(jnp.dot(t, aw1[...], preferred_element_type=_f32)
                   + ab1[...])
        logits[...] = jnp.dot(a, aw2[...], preferred_element_type=_f32)             + ab2[...]
        cv = _leaky(jnp.dot(t, cw1[...], preferred_element_type=_f32)
                    + cb1[...])
        value8[...] = jnp.dot(cv, cw2[...], preferred_element_type=_f32)             + cb2[...]


def _pool_heads(h, p, twt, tb, aw1, ab1, aw2, ab2, cw1, cb1, cw2, cb2):
    grid = (NB,)
    bs_rows = pl.BlockSpec((BR, D), lambda b: (b, 0))
    bs_p = pl.BlockSpec((BR, G), lambda b: (b, 0))
    bs_w = pl.BlockSpec((D, D), lambda b: (0, 0))
    bs_g = pl.BlockSpec((G, D), lambda b: (0, 0))
    bs_wa = pl.BlockSpec((D, NUM_ACTIONS), lambda b: (0, 0))
    bs_ga = pl.BlockSpec((G, NUM_ACTIONS), lambda b: (0, 0))
    bs_wc = pl.BlockSpec((D, G), lambda b: (0, 0))
    bs_gc = pl.BlockSpec((G, G), lambda b: (0, 0))
    return pl.pallas_call(
        _pool_heads_body,
        grid=grid,
        in_specs=[bs_rows, bs_p, bs_w, bs_g, bs_w, bs_g, bs_wa, bs_ga,
                  bs_w, bs_g, bs_wc, bs_gc],
        out_specs=[bs_ga, bs_gc],
        out_shape=[
            jax.ShapeDtypeStruct((G, NUM_ACTIONS), _f32),
            jax.ShapeDtypeStruct((G, G), _f32),
        ],
        scratch_shapes=[
            pltpu.VMEM((G, D), _f32),
            pltpu.VMEM((G, D), _f32),
        ],
    )(h, p, twt, tb, aw1, ab1, aw2, ab2, cw1, cb1, cw2, cb2)


def kernel(x, params, edge_index, batch):
    i32 = jnp.int32
    src = edge_index[0]
    dst = edge_index[1]
    pad_e = jnp.full((EPAD - E,), N, dtype=i32)
    src2 = jnp.concatenate([src.astype(i32), pad_e]).reshape(EROWS, CHUNK)
    dst2 = jnp.concatenate([dst.astype(i32), pad_e]).reshape(EROWS, CHUNK)

    h = jnp.pad(x.astype(_f32), ((0, NPAD - N), (0, 0)))
    batch_pad = jnp.pad(batch.astype(i32), (0, NPAD - N), constant_values=G)
    p = (batch_pad[:, None] == jnp.arange(G, dtype=i32)[None, :]).astype(_f32)

    z_d = jnp.zeros((RPT, D), _f32)
    o_d = jnp.ones((CHUNK, D), _f32)

    bcast = lambda v, w=D: jnp.broadcast_to(v.reshape(1, -1), (G, w))

    cf = cb = None
    for i in range(NUM_LAYERS):
        outs = _sc_aggregate(h, src2, dst2, z_d, o_d, with_counts=(i == 0))
        if i == 0:
            sf, sb, cf, cb = outs
        else:
            sf, sb = outs
        wlt = params['conv_Wl'][i].T
        wrt = params['conv_Wr'][i].T
        bl8 = bcast(params['conv_bl'][i])
        hnew, s1, s2, cg = _layer_a(h, sf, sb, cf, cb, wlt, wrt, bl8, p)
        h = _layer_b(hnew, h, s1, s2, cg, p,
                     bcast(params['norm_w'][i]),
                     bcast(params['norm_b'][i]),
                     bcast(params['norm_ms'][i]),
                     residual=(i >= RESIDUAL_START))

    cw2 = jnp.pad(params['critic_W2'].T, ((0, 0), (0, G - 1)))
    cb2 = jnp.broadcast_to(params['critic_b2'].reshape(1, 1), (G, G))
    logits, value8 = _pool_heads(
        h, p,
        params['trunk_W'].T, bcast(params['trunk_b']),
        params['actor_W1'].T, bcast(params['actor_b1']),
        params['actor_W2'].T, bcast(params['actor_b2'], NUM_ACTIONS),
        params['critic_W1'].T, bcast(params['critic_b1']),
        cw2, cb2)
    return logits, value8[:, :1]


# speedup vs baseline: 1.6036x; 1.0025x over previous
import functools

import jax
import jax.numpy as jnp
from jax import lax
from jax.experimental import pallas as pl
from jax.experimental.pallas import tpu as pltpu
from jax.experimental.pallas import tpu_sc as plsc

N = 10000
E = 320000
D = 128
H = 128
G = 8
NUM_ACTIONS = 64
NUM_LAYERS = 4
RESIDUAL_START = 2
NEG_SLOPE = 0.01
EPS = 1e-5

NS = 16
CHUNK = 128
CH_PER_TILE = 160
EPT = CHUNK * CH_PER_TILE
EPAD = EPT * NS
EROWS = EPAD // CHUNK
NPAD = 10240
NHALF = NPAD // 2
RPT = NHALF // NS
CW = 16
CH_BUF = 40
NRING = 4
IGN = -1

_f32 = jnp.float32
_HI = lax.Precision.HIGHEST


def _leaky(v):
    return jnp.where(v >= 0, v, NEG_SLOPE * v)


def _make_sc_dir():
    mesh = plsc.VectorSubcoreMesh(core_axis_name="c", subcore_axis_name="s",
                                  num_cores=2, num_subcores=NS)

    out_type = [jax.ShapeDtypeStruct((NPAD, D), _f32)]

    scratch = [
        pltpu.VMEM((CH_BUF, CHUNK), jnp.int32),
        pltpu.VMEM((CH_BUF, CHUNK), jnp.int32),
        pltpu.VMEM((NRING, CHUNK), jnp.int32),
    ] + [pltpu.VMEM((CHUNK, D), _f32) for _ in range(NRING)] \
      + [pltpu.SemaphoreType.DMA for _ in range(2 * NRING)] + [
        pltpu.VMEM_SHARED((NHALF, D), _f32),
    ]

    def body(h_hbm, gidx2, sidx2, z_d, sum_o, gidx, sidx, sloc, *rest):
        rows = rest[:NRING]
        gsem = rest[NRING:2 * NRING]
        ssem = rest[2 * NRING:3 * NRING]
        acc = rest[3 * NRING]
        c = lax.axis_index("c")
        s = lax.axis_index("s")
        nbase = c * NHALF

        pltpu.sync_copy(z_d, acc.at[pl.ds(s * RPT, RPT)])

        plsc.subcore_barrier()

        def fire_g(j, k):
            pltpu.async_copy(h_hbm.at[gidx.at[j]], rows[k], gsem[k])

        def wait_g(k):
            pltpu.make_async_copy(h_hbm.at[gidx.at[0]], rows[k],
                                  gsem[k]).wait()

        def rewrite(j, k):
            for q in range(CHUNK // 16):
                t = sidx[j, pl.ds(q * 16, 16)]
                ok = jnp.logical_and(t >= nbase, t < nbase + NHALF)
                sloc[k, pl.ds(q * 16, 16)] = jnp.where(ok, t - nbase, IGN)

        def fire_s(k):
            pltpu.async_copy(
                rows[k], acc.at[plsc.Indices(sloc.at[k], ignored_value=IGN)],
                ssem[k], add=True)

        def wait_s(k):
            pltpu.make_async_copy(
                rows[k], acc.at[plsc.Indices(sloc.at[k], ignored_value=IGN)],
                ssem[k]).wait()

        base = s * CH_PER_TILE
        ngroups = CH_BUF // NRING
        for piece in range(CH_PER_TILE // CH_BUF):
            hb = base + piece * CH_BUF
            pltpu.sync_copy(gidx2.at[pl.ds(hb, CH_BUF)], gidx)
            pltpu.sync_copy(sidx2.at[pl.ds(hb, CH_BUF)], sidx)

            for k in range(NRING):
                fire_g(k, k)

            def group(m, carry):
                for k in range(NRING):
                    wait_g(k)
                    rewrite(m * NRING + k, k)
                    fire_s(k)
                for k in range(NRING):
                    wait_s(k)
                    fire_g((m + 1) * NRING + k, k)
                return carry

            lax.fori_loop(0, ngroups - 1, group, 0)
            mlast = ngroups - 1
            for k in range(NRING):
                wait_g(k)
                rewrite(mlast * NRING + k, k)
                fire_s(k)
            for k in range(NRING):
                wait_s(k)

        plsc.subcore_barrier()

        loc = pl.ds(s * RPT, RPT)
        glob = pl.ds(nbase + s * RPT, RPT)
        pltpu.sync_copy(acc.at[loc], sum_o.at[glob])

    return pl.kernel(body, out_type=out_type, mesh=mesh,
                     scratch_types=scratch)


def _make_sc_cnt():
    mesh = plsc.VectorSubcoreMesh(core_axis_name="c", subcore_axis_name="s",
                                  num_cores=2, num_subcores=NS)

    out_type = [jax.ShapeDtypeStruct((NPAD, D), _f32)]

    scratch = [
        pltpu.VMEM((CH_PER_TILE, CHUNK), jnp.int32),
        pltpu.VMEM((CHUNK,), jnp.int32),
        pltpu.VMEM((CHUNK, D), _f32),
        pltpu.VMEM_SHARED((NHALF, D), _f32),
    ]

    def body(sidx2, z_d, o_d, cnt_o, sidx, sloc, ones_v, cnt):
        c = lax.axis_index("c")
        s = lax.axis_index("s")
        nbase = c * NHALF

        pltpu.sync_copy(z_d, cnt.at[pl.ds(s * RPT, RPT)])
        pltpu.sync_copy(o_d, ones_v)
        base = s * CH_PER_TILE
        pltpu.sync_copy(sidx2.at[pl.ds(base, CH_PER_TILE)], sidx)

        plsc.subcore_barrier()

        def step(j, carry):
            for k in range(CHUNK // 16):
                t = sidx[j, pl.ds(k * 16, 16)]
                ok = jnp.logical_and(t >= nbase, t < nbase + NHALF)
                sloc[pl.ds(k * 16, 16)] = jnp.where(ok, t - nbase, IGN)
            idx = plsc.Indices(sloc, ignored_value=IGN)
            pltpu.sync_copy(ones_v, cnt.at[idx], add=True)
            return carry

        lax.fori_loop(0, CH_PER_TILE, step, 0)

        plsc.subcore_barrier()

        loc = pl.ds(s * RPT, RPT)
        glob = pl.ds(nbase + s * RPT, RPT)
        pltpu.sync_copy(cnt.at[loc], cnt_o.at[glob])

    return pl.kernel(body, out_type=out_type, mesh=mesh,
                     scratch_types=scratch)


_sc_cache = {}


def _sc_aggregate(h, src2, dst2, z_d, o_d, with_counts):
    if "dir" not in _sc_cache:
        _sc_cache["dir"] = _make_sc_dir()
        _sc_cache["cnt"] = _make_sc_cnt()
    sf, = _sc_cache["dir"](h, src2, dst2, z_d)
    sb, = _sc_cache["dir"](h, dst2, src2, z_d)
    if with_counts:
        cf, = _sc_cache["cnt"](dst2, z_d, o_d)
        cb, = _sc_cache["cnt"](src2, z_d, o_d)
        return sf, sb, cf, cb
    return sf, sb


BR = 512
NB = NPAD // BR


def _layer_a_body(h, sf, sb, cf, cb, wlt, wrt, bl, p,
                  hnew, s1, s2, cg):
    icd = 1.0 / jnp.maximum(cf[...], 1.0)
    ics = 1.0 / jnp.maximum(cb[...], 1.0)
    aggc = 0.5 * (sf[...] * icd + sb[...] * ics)
    hn = (jnp.dot(aggc, wlt[...], preferred_element_type=_f32)
          + jnp.dot(h[...], wrt[...], preferred_element_type=_f32)
          + bl[0:1, :])
    hnew[...] = hn
    pt = p[...]
    dn = (((0,), (0,)), ((), ()))
    s1c = lax.dot_general(pt, hn, dn, preferred_element_type=_f32,
                          precision=_HI)
    s2c = lax.dot_general(pt, hn * hn, dn, preferred_element_type=_f32,
                          precision=_HI)
    cgc = lax.dot_general(pt, jnp.ones_like(hn), dn,
                          preferred_element_type=_f32, precision=_HI)

    @pl.when(pl.program_id(0) == 0)
    def _():
        s1[...] = s1c
        s2[...] = s2c
        cg[...] = cgc

    @pl.when(pl.program_id(0) > 0)
    def _():
        s1[...] += s1c
        s2[...] += s2c
        cg[...] += cgc


def _layer_a(h, sf, sb, cf, cb, wlt, wrt, bl8, p):
    grid = (NB,)
    bs_rows = pl.BlockSpec((BR, D), lambda b: (b, 0))
    bs_w = pl.BlockSpec((D, D), lambda b: (0, 0))
    bs_g = pl.BlockSpec((G, D), lambda b: (0, 0))
    bs_p = pl.BlockSpec((BR, G), lambda b: (b, 0))
    return pl.pallas_call(
        _layer_a_body,
        grid=grid,
        in_specs=[bs_rows, bs_rows, bs_rows, bs_rows, bs_rows,
                  bs_w, bs_w, bs_g, bs_p],
        out_specs=[bs_rows, bs_g, bs_g, bs_g],
        out_shape=[
            jax.ShapeDtypeStruct((NPAD, D), _f32),
            jax.ShapeDtypeStruct((G, D), _f32),
            jax.ShapeDtypeStruct((G, D), _f32),
            jax.ShapeDtypeStruct((G, D), _f32),
        ],
    )(h, sf, sb, cf, cb, wlt, wrt, bl8, p)


def _make_layer_b_body(residual: bool):
    def body(*args):
        if residual:
            (hnew, hprev, s1, s2, cg, p, w8, b8, ms8, hout) = args
        else:
            (hnew, s1, s2, cg, p, w8, b8, ms8, hout) = args
        cgv = jnp.maximum(cg[...], 1.0)
        mean = s1[...] / cgv
        ms = ms8[...]
        var = s2[...] / cgv - mean * mean * ms * (2.0 - ms)
        rstd = lax.rsqrt(var + EPS)
        scale = w8[...] * rstd
        shift = b8[...] - scale * ms * mean
        rowscale = jnp.dot(p[...], scale, preferred_element_type=_f32,
                           precision=_HI)
        rowshift = jnp.dot(p[...], shift, preferred_element_type=_f32,
                           precision=_HI)
        v = _leaky(hnew[...] * rowscale + rowshift)
        if residual:
            v = v + hprev[...]
        hout[...] = v
    return body


def _layer_b(hnew, hprev, s1, s2, cg, p, w8, b8, ms8, residual):
    grid = (NB,)
    bs_rows = pl.BlockSpec((BR, D), lambda b: (b, 0))
    bs_g = pl.BlockSpec((G, D), lambda b: (0, 0))
    bs_p = pl.BlockSpec((BR, G), lambda b: (b, 0))
    in_specs = [bs_rows] + ([bs_rows] if residual else []) + \
               [bs_g, bs_g, bs_g, bs_p, bs_g, bs_g, bs_g]
    args = (hnew,) + ((hprev,) if residual else ()) + \
           (s1, s2, cg, p, w8, b8, ms8)
    return pl.pallas_call(
        _make_layer_b_body(residual),
        grid=grid,
        in_specs=in_specs,
        out_specs=bs_rows,
        out_shape=jax.ShapeDtypeStruct((NPAD, D), _f32),
    )(*args)


def _pool_heads_body(h, p, twt, tb, aw1, ab1, aw2, ab2, cw1, cb1, cw2, cb2,
                     logits, value8, acc_e, acc_c):
    b = pl.program_id(0)
    pt = p[...]
    dn = (((0,), (0,)), ((), ()))
    e = lax.dot_general(pt, h[...], dn, preferred_element_type=_f32,
                        precision=_HI)
    c = lax.dot_general(pt, jnp.ones_like(h[...]), dn,
                        preferred_element_type=_f32, precision=_HI)

    @pl.when(b == 0)
    def _():
        acc_e[...] = e
        acc_c[...] = c

    @pl.when(b > 0)
    def _():
        acc_e[...] += e
        acc_c[...] += c

    @pl.when(b == NB - 1)
    def _():
        emb = acc_e[...] / jnp.maximum(acc_c[...], 1.0)
        t = _leaky(jnp.dot(emb, twt[...], preferred_element_type=_f32)
                   + tb[...])
        a = _leaky(jnp.dot(t, aw1[...], preferred_element_type=_f32)
                   + ab1[...])
        logits[...] = jnp.dot(a, aw2[...], preferred_element_type=_f32)             + ab2[...]
        cv = _leaky(jnp.dot(t, cw1[...], preferred_element_type=_f32)
                    + cb1[...])
        value8[...] = jnp.dot(cv, cw2[...], preferred_element_type=_f32)             + cb2[...]


def _pool_heads(h, p, twt, tb, aw1, ab1, aw2, ab2, cw1, cb1, cw2, cb2):
    grid = (NB,)
    bs_rows = pl.BlockSpec((BR, D), lambda b: (b, 0))
    bs_p = pl.BlockSpec((BR, G), lambda b: (b, 0))
    bs_w = pl.BlockSpec((D, D), lambda b: (0, 0))
    bs_g = pl.BlockSpec((G, D), lambda b: (0, 0))
    bs_wa = pl.BlockSpec((D, NUM_ACTIONS), lambda b: (0, 0))
    bs_ga = pl.BlockSpec((G, NUM_ACTIONS), lambda b: (0, 0))
    bs_wc = pl.BlockSpec((D, G), lambda b: (0, 0))
    bs_gc = pl.BlockSpec((G, G), lambda b: (0, 0))
    return pl.pallas_call(
        _pool_heads_body,
        grid=grid,
        in_specs=[bs_rows, bs_p, bs_w, bs_g, bs_w, bs_g, bs_wa, bs_ga,
                  bs_w, bs_g, bs_wc, bs_gc],
        out_specs=[bs_ga, bs_gc],
        out_shape=[
            jax.ShapeDtypeStruct((G, NUM_ACTIONS), _f32),
            jax.ShapeDtypeStruct((G, G), _f32),
        ],
        scratch_shapes=[
            pltpu.VMEM((G, D), _f32),
            pltpu.VMEM((G, D), _f32),
        ],
    )(h, p, twt, tb, aw1, ab1, aw2, ab2, cw1, cb1, cw2, cb2)


def kernel(x, params, edge_index, batch):
    i32 = jnp.int32
    src = edge_index[0]
    dst = edge_index[1]
    pad_e = jnp.full((EPAD - E,), N, dtype=i32)
    src2 = jnp.concatenate([src.astype(i32), pad_e]).reshape(EROWS, CHUNK)
    dst2 = jnp.concatenate([dst.astype(i32), pad_e]).reshape(EROWS, CHUNK)

    h = jnp.pad(x.astype(_f32), ((0, NPAD - N), (0, 0)))
    batch_pad = jnp.pad(batch.astype(i32), (0, NPAD - N), constant_values=G)
    p = (batch_pad[:, None] == jnp.arange(G, dtype=i32)[None, :]).astype(_f32)

    z_d = jnp.zeros((RPT, D), _f32)
    o_d = jnp.ones((CHUNK, D), _f32)

    bcast = lambda v, w=D: jnp.broadcast_to(v.reshape(1, -1), (G, w))

    cf = cb = None
    for i in range(NUM_LAYERS):
        outs = _sc_aggregate(h, src2, dst2, z_d, o_d, with_counts=(i == 0))
        if i == 0:
            sf, sb, cf, cb = outs
        else:
            sf, sb = outs
        wlt = params['conv_Wl'][i].T
        wrt = params['conv_Wr'][i].T
        bl8 = bcast(params['conv_bl'][i])
        hnew, s1, s2, cg = _layer_a(h, sf, sb, cf, cb, wlt, wrt, bl8, p)
        h = _layer_b(hnew, h, s1, s2, cg, p,
                     bcast(params['norm_w'][i]),
                     bcast(params['norm_b'][i]),
                     bcast(params['norm_ms'][i]),
                     residual=(i >= RESIDUAL_START))

    cw2 = jnp.pad(params['critic_W2'].T, ((0, 0), (0, G - 1)))
    cb2 = jnp.broadcast_to(params['critic_b2'].reshape(1, 1), (G, G))
    logits, value8 = _pool_heads(
        h, p,
        params['trunk_W'].T, bcast(params['trunk_b']),
        params['actor_W1'].T, bcast(params['actor_b1']),
        params['actor_W2'].T, bcast(params['actor_b2'], NUM_ACTIONS),
        params['critic_W1'].T, bcast(params['critic_b1']),
        cw2, cb2)
    return logits, value8[:, :1]


# speedup vs baseline: 2.4850x; 1.5496x over previous
import functools

import jax
import jax.numpy as jnp
from jax import lax
from jax.experimental import pallas as pl
from jax.experimental.pallas import tpu as pltpu
from jax.experimental.pallas import tpu_sc as plsc

N = 10000
E = 320000
D = 128
H = 128
G = 8
NUM_ACTIONS = 64
NUM_LAYERS = 4
RESIDUAL_START = 2
NEG_SLOPE = 0.01
EPS = 1e-5

NS = 16
CHUNK = 128
CH_PER_TILE = 160
EPT = CHUNK * CH_PER_TILE
EPAD = EPT * NS
EROWS = EPAD // CHUNK
NPAD = 10240
NHALF = NPAD // 2
RPT = NHALF // NS
CW = 16
CH_BUF = 40
NRING = 4
IGN = -1

_f32 = jnp.float32
_HI = lax.Precision.HIGHEST


def _leaky(v):
    return jnp.where(v >= 0, v, NEG_SLOPE * v)


def _make_sc_dir():
    mesh = plsc.VectorSubcoreMesh(core_axis_name="c", subcore_axis_name="s",
                                  num_cores=2, num_subcores=NS)

    out_type = [jax.ShapeDtypeStruct((NPAD, D), _f32)]

    scratch = [
        pltpu.VMEM((CH_BUF, CHUNK), jnp.int32),
        pltpu.VMEM((CH_BUF, CHUNK), jnp.int32),
        pltpu.VMEM((NRING, CHUNK), jnp.int32),
        pltpu.VMEM((NRING, CHUNK), jnp.int32),
    ] + [pltpu.VMEM((CHUNK, D), _f32) for _ in range(NRING)] \
      + [pltpu.SemaphoreType.DMA for _ in range(2 * NRING)] + [
        pltpu.VMEM_SHARED((NHALF, D), _f32),
    ]

    def body(h_hbm, gidx2, sidx2, z_d, sum_o, gidx, sidx, gloc, sloc, *rest):
        rows = rest[:NRING]
        gsem = rest[NRING:2 * NRING]
        ssem = rest[2 * NRING:3 * NRING]
        acc = rest[3 * NRING]
        c = lax.axis_index("c")
        s = lax.axis_index("s")
        nbase = c * NHALF

        pltpu.sync_copy(z_d, acc.at[pl.ds(s * RPT, RPT)])

        plsc.subcore_barrier()

        def rewrite(j, k):
            for q in range(CHUNK // 16):
                sl = pl.ds(q * 16, 16)
                t = sidx[j, sl]
                g = gidx[j, sl]
                ok = jnp.logical_and(t >= nbase, t < nbase + NHALF)
                gloc[k, sl] = jnp.where(ok, g, IGN)
                sloc[k, sl] = jnp.where(ok, t - nbase, IGN)

        def fire_g(k):
            pltpu.async_copy(
                h_hbm.at[plsc.Indices(gloc.at[k], ignored_value=IGN)],
                rows[k], gsem[k])

        def wait_g(k):
            pltpu.make_async_copy(
                h_hbm.at[plsc.Indices(gloc.at[k], ignored_value=IGN)],
                rows[k], gsem[k]).wait()

        def fire_s(k):
            pltpu.async_copy(
                rows[k], acc.at[plsc.Indices(sloc.at[k], ignored_value=IGN)],
                ssem[k], add=True)

        def wait_s(k):
            pltpu.make_async_copy(
                rows[k], acc.at[plsc.Indices(sloc.at[k], ignored_value=IGN)],
                ssem[k]).wait()

        base = s * CH_PER_TILE
        ngroups = CH_BUF // NRING
        for piece in range(CH_PER_TILE // CH_BUF):
            hb = base + piece * CH_BUF
            pltpu.sync_copy(gidx2.at[pl.ds(hb, CH_BUF)], gidx)
            pltpu.sync_copy(sidx2.at[pl.ds(hb, CH_BUF)], sidx)

            for k in range(NRING):
                rewrite(k, k)
                fire_g(k)

            def group(m, carry):
                for k in range(NRING):
                    wait_g(k)
                    fire_s(k)
                for k in range(NRING):
                    wait_s(k)
                    rewrite((m + 1) * NRING + k, k)
                    fire_g(k)
                return carry

            lax.fori_loop(0, ngroups - 1, group, 0)
            for k in range(NRING):
                wait_g(k)
                fire_s(k)
            for k in range(NRING):
                wait_s(k)

        plsc.subcore_barrier()

        loc = pl.ds(s * RPT, RPT)
        glob = pl.ds(nbase + s * RPT, RPT)
        pltpu.sync_copy(acc.at[loc], sum_o.at[glob])

    return pl.kernel(body, out_type=out_type, mesh=mesh,
                     scratch_types=scratch)


def _make_sc_cnt():
    mesh = plsc.VectorSubcoreMesh(core_axis_name="c", subcore_axis_name="s",
                                  num_cores=2, num_subcores=NS)

    out_type = [jax.ShapeDtypeStruct((NPAD, D), _f32)]

    scratch = [
        pltpu.VMEM((CH_PER_TILE, CHUNK), jnp.int32),
        pltpu.VMEM((CHUNK,), jnp.int32),
        pltpu.VMEM((CHUNK, D), _f32),
        pltpu.VMEM_SHARED((NHALF, D), _f32),
    ]

    def body(sidx2, z_d, o_d, cnt_o, sidx, sloc, ones_v, cnt):
        c = lax.axis_index("c")
        s = lax.axis_index("s")
        nbase = c * NHALF

        pltpu.sync_copy(z_d, cnt.at[pl.ds(s * RPT, RPT)])
        pltpu.sync_copy(o_d, ones_v)
        base = s * CH_PER_TILE
        pltpu.sync_copy(sidx2.at[pl.ds(base, CH_PER_TILE)], sidx)

        plsc.subcore_barrier()

        def step(j, carry):
            for k in range(CHUNK // 16):
                t = sidx[j, pl.ds(k * 16, 16)]
                ok = jnp.logical_and(t >= nbase, t < nbase + NHALF)
                sloc[pl.ds(k * 16, 16)] = jnp.where(ok, t - nbase, IGN)
            idx = plsc.Indices(sloc, ignored_value=IGN)
            pltpu.sync_copy(ones_v, cnt.at[idx], add=True)
            return carry

        lax.fori_loop(0, CH_PER_TILE, step, 0)

        plsc.subcore_barrier()

        loc = pl.ds(s * RPT, RPT)
        glob = pl.ds(nbase + s * RPT, RPT)
        pltpu.sync_copy(cnt.at[loc], cnt_o.at[glob])

    return pl.kernel(body, out_type=out_type, mesh=mesh,
                     scratch_types=scratch)


_sc_cache = {}


def _sc_aggregate(h, src2, dst2, z_d, o_d, with_counts):
    if "dir" not in _sc_cache:
        _sc_cache["dir"] = _make_sc_dir()
        _sc_cache["cnt"] = _make_sc_cnt()
    sf, = _sc_cache["dir"](h, src2, dst2, z_d)
    sb, = _sc_cache["dir"](h, dst2, src2, z_d)
    if with_counts:
        cf, = _sc_cache["cnt"](dst2, z_d, o_d)
        cb, = _sc_cache["cnt"](src2, z_d, o_d)
        return sf, sb, cf, cb
    return sf, sb


BR = 512
NB = NPAD // BR


def _layer_a_body(h, sf, sb, cf, cb, wlt, wrt, bl, p,
                  hnew, s1, s2, cg):
    icd = 1.0 / jnp.maximum(cf[...], 1.0)
    ics = 1.0 / jnp.maximum(cb[...], 1.0)
    aggc = 0.5 * (sf[...] * icd + sb[...] * ics)
    hn = (jnp.dot(aggc, wlt[...], preferred_element_type=_f32)
          + jnp.dot(h[...], wrt[...], preferred_element_type=_f32)
          + bl[0:1, :])
    hnew[...] = hn
    pt = p[...]
    dn = (((0,), (0,)), ((), ()))
    s1c = lax.dot_general(pt, hn, dn, preferred_element_type=_f32,
                          precision=_HI)
    s2c = lax.dot_general(pt, hn * hn, dn, preferred_element_type=_f32,
                          precision=_HI)
    cgc = lax.dot_general(pt, jnp.ones_like(hn), dn,
                          preferred_element_type=_f32, precision=_HI)

    @pl.when(pl.program_id(0) == 0)
    def _():
        s1[...] = s1c
        s2[...] = s2c
        cg[...] = cgc

    @pl.when(pl.program_id(0) > 0)
    def _():
        s1[...] += s1c
        s2[...] += s2c
        cg[...] += cgc


def _layer_a(h, sf, sb, cf, cb, wlt, wrt, bl8, p):
    grid = (NB,)
    bs_rows = pl.BlockSpec((BR, D), lambda b: (b, 0))
    bs_w = pl.BlockSpec((D, D), lambda b: (0, 0))
    bs_g = pl.BlockSpec((G, D), lambda b: (0, 0))
    bs_p = pl.BlockSpec((BR, G), lambda b: (b, 0))
    return pl.pallas_call(
        _layer_a_body,
        grid=grid,
        in_specs=[bs_rows, bs_rows, bs_rows, bs_rows, bs_rows,
                  bs_w, bs_w, bs_g, bs_p],
        out_specs=[bs_rows, bs_g, bs_g, bs_g],
        out_shape=[
            jax.ShapeDtypeStruct((NPAD, D), _f32),
            jax.ShapeDtypeStruct((G, D), _f32),
            jax.ShapeDtypeStruct((G, D), _f32),
            jax.ShapeDtypeStruct((G, D), _f32),
        ],
    )(h, sf, sb, cf, cb, wlt, wrt, bl8, p)


def _make_layer_b_body(residual: bool):
    def body(*args):
        if residual:
            (hnew, hprev, s1, s2, cg, p, w8, b8, ms8, hout) = args
        else:
            (hnew, s1, s2, cg, p, w8, b8, ms8, hout) = args
        cgv = jnp.maximum(cg[...], 1.0)
        mean = s1[...] / cgv
        ms = ms8[...]
        var = s2[...] / cgv - mean * mean * ms * (2.0 - ms)
        rstd = lax.rsqrt(var + EPS)
        scale = w8[...] * rstd
        shift = b8[...] - scale * ms * mean
        rowscale = jnp.dot(p[...], scale, preferred_element_type=_f32,
                           precision=_HI)
        rowshift = jnp.dot(p[...], shift, preferred_element_type=_f32,
                           precision=_HI)
        v = _leaky(hnew[...] * rowscale + rowshift)
        if residual:
            v = v + hprev[...]
        hout[...] = v
    return body


def _layer_b(hnew, hprev, s1, s2, cg, p, w8, b8, ms8, residual):
    grid = (NB,)
    bs_rows = pl.BlockSpec((BR, D), lambda b: (b, 0))
    bs_g = pl.BlockSpec((G, D), lambda b: (0, 0))
    bs_p = pl.BlockSpec((BR, G), lambda b: (b, 0))
    in_specs = [bs_rows] + ([bs_rows] if residual else []) + \
               [bs_g, bs_g, bs_g, bs_p, bs_g, bs_g, bs_g]
    args = (hnew,) + ((hprev,) if residual else ()) + \
           (s1, s2, cg, p, w8, b8, ms8)
    return pl.pallas_call(
        _make_layer_b_body(residual),
        grid=grid,
        in_specs=in_specs,
        out_specs=bs_rows,
        out_shape=jax.ShapeDtypeStruct((NPAD, D), _f32),
    )(*args)


def _pool_heads_body(h, p, twt, tb, aw1, ab1, aw2, ab2, cw1, cb1, cw2, cb2,
                     logits, value8, acc_e, acc_c):
    b = pl.program_id(0)
    pt = p[...]
    dn = (((0,), (0,)), ((), ()))
    e = lax.dot_general(pt, h[...], dn, preferred_element_type=_f32,
                        precision=_HI)
    c = lax.dot_general(pt, jnp.ones_like(h[...]), dn,
                        preferred_element_type=_f32, precision=_HI)

    @pl.when(b == 0)
    def _():
        acc_e[...] = e
        acc_c[...] = c

    @pl.when(b > 0)
    def _():
        acc_e[...] += e
        acc_c[...] += c

    @pl.when(b == NB - 1)
    def _():
        emb = acc_e[...] / jnp.maximum(acc_c[...], 1.0)
        t = _leaky(jnp.dot(emb, twt[...], preferred_element_type=_f32)
                   + tb[...])
        a = _leaky(jnp.dot(t, aw1[...], preferred_element_type=_f32)
                   + ab1[...])
        logits[...] = jnp.dot(a, aw2[...], preferred_element_type=_f32)             + ab2[...]
        cv = _leaky(jnp.dot(t, cw1[...], preferred_element_type=_f32)
                    + cb1[...])
        value8[...] = jnp.dot(cv, cw2[...], preferred_element_type=_f32)             + cb2[...]


def _pool_heads(h, p, twt, tb, aw1, ab1, aw2, ab2, cw1, cb1, cw2, cb2):
    grid = (NB,)
    bs_rows = pl.BlockSpec((BR, D), lambda b: (b, 0))
    bs_p = pl.BlockSpec((BR, G), lambda b: (b, 0))
    bs_w = pl.BlockSpec((D, D), lambda b: (0, 0))
    bs_g = pl.BlockSpec((G, D), lambda b: (0, 0))
    bs_wa = pl.BlockSpec((D, NUM_ACTIONS), lambda b: (0, 0))
    bs_ga = pl.BlockSpec((G, NUM_ACTIONS), lambda b: (0, 0))
    bs_wc = pl.BlockSpec((D, G), lambda b: (0, 0))
    bs_gc = pl.BlockSpec((G, G), lambda b: (0, 0))
    return pl.pallas_call(
        _pool_heads_body,
        grid=grid,
        in_specs=[bs_rows, bs_p, bs_w, bs_g, bs_w, bs_g, bs_wa, bs_ga,
                  bs_w, bs_g, bs_wc, bs_gc],
        out_specs=[bs_ga, bs_gc],
        out_shape=[
            jax.ShapeDtypeStruct((G, NUM_ACTIONS), _f32),
            jax.ShapeDtypeStruct((G, G), _f32),
        ],
        scratch_shapes=[
            pltpu.VMEM((G, D), _f32),
            pltpu.VMEM((G, D), _f32),
        ],
    )(h, p, twt, tb, aw1, ab1, aw2, ab2, cw1, cb1, cw2, cb2)


def kernel(x, params, edge_index, batch):
    i32 = jnp.int32
    src = edge_index[0]
    dst = edge_index[1]
    pad_e = jnp.full((EPAD - E,), N, dtype=i32)
    src2 = jnp.concatenate([src.astype(i32), pad_e]).reshape(EROWS, CHUNK)
    dst2 = jnp.concatenate([dst.astype(i32), pad_e]).reshape(EROWS, CHUNK)

    h = jnp.pad(x.astype(_f32), ((0, NPAD - N), (0, 0)))
    batch_pad = jnp.pad(batch.astype(i32), (0, NPAD - N), constant_values=G)
    p = (batch_pad[:, None] == jnp.arange(G, dtype=i32)[None, :]).astype(_f32)

    z_d = jnp.zeros((RPT, D), _f32)
    o_d = jnp.ones((CHUNK, D), _f32)

    bcast = lambda v, w=D: jnp.broadcast_to(v.reshape(1, -1), (G, w))

    cf = cb = None
    for i in range(NUM_LAYERS):
        outs = _sc_aggregate(h, src2, dst2, z_d, o_d, with_counts=(i == 0))
        if i == 0:
            sf, sb, cf, cb = outs
        else:
            sf, sb = outs
        wlt = params['conv_Wl'][i].T
        wrt = params['conv_Wr'][i].T
        bl8 = bcast(params['conv_bl'][i])
        hnew, s1, s2, cg = _layer_a(h, sf, sb, cf, cb, wlt, wrt, bl8, p)
        h = _layer_b(hnew, h, s1, s2, cg, p,
                     bcast(params['norm_w'][i]),
                     bcast(params['norm_b'][i]),
                     bcast(params['norm_ms'][i]),
                     residual=(i >= RESIDUAL_START))

    cw2 = jnp.pad(params['critic_W2'].T, ((0, 0), (0, G - 1)))
    cb2 = jnp.broadcast_to(params['critic_b2'].reshape(1, 1), (G, G))
    logits, value8 = _pool_heads(
        h, p,
        params['trunk_W'].T, bcast(params['trunk_b']),
        params['actor_W1'].T, bcast(params['actor_b1']),
        params['actor_W2'].T, bcast(params['actor_b2'], NUM_ACTIONS),
        params['critic_W1'].T, bcast(params['critic_b1']),
        cw2, cb2)
    return logits, value8[:, :1]
